# baseline (device time: 144110 ns/iter reference)
import contextlib

import jax
import jax.numpy as jnp
from jax import lax
from jax.experimental import pallas as pl
from jax.experimental.pallas import tpu as pltpu

PROFILE_SCOPES = False


def _scope(name):
    return jax.named_scope(name) if PROFILE_SCOPES else contextlib.nullcontext()


N_DEV = 4
M_BLK = 1024
K_BLK = 1024
N_OUT = 8192
NT = 512
T = N_OUT // NT
WDEPTH = 3
QT = 1024
QTN = N_OUT // QT


def kernel(x, w_mat):

    def body(x_hbm, w_hbm, out_hbm, acc, xf, xs, xg, xb, wbuf, amax_my,
             amax_all, send_sems, recv_sems, a_send_sems, a_recv_sems,
             w_sems, out_sems, loc_sem, stage_sem):
        my = lax.axis_index("i")

        barrier_sem = pltpu.get_barrier_semaphore()
        for d in range(1, N_DEV):
            pl.semaphore_signal(
                barrier_sem, inc=1,
                device_id=((my + d) % N_DEV,),
                device_id_type=pl.DeviceIdType.MESH,
            )
        pl.semaphore_wait(barrier_sem, N_DEV - 1)

        os_order = (0, 3, 1, 2)
        flat = [(o, t) for o in os_order for t in range(T)]

        def w_dma(f):
            o, t = flat[f]
            j = (my + o) % N_DEV
            return pltpu.make_async_copy(
                w_hbm.at[pl.ds(j * K_BLK, K_BLK), pl.ds(t * NT, NT)],
                wbuf.at[f % WDEPTH], w_sems.at[f % WDEPTH])

        w_dma(0).start()
        w_dma(1).start()
        loc = pltpu.make_async_copy(
            x_hbm.at[pl.ds(my * M_BLK, M_BLK), :], xf, loc_sem)
        loc.start()

        sends = []
        stage_dmas = {}

        def stage_start(d):
            r = (my + d) % N_DEV
            cp = pltpu.make_async_copy(
                x_hbm.at[pl.ds(r * M_BLK, M_BLK), :], xb, stage_sem)
            cp.start()
            stage_dmas[d] = cp

        def stage_finish(d):
            stage_dmas[d].wait()
            xs[d - 1] = xb[:, :].astype(jnp.bfloat16)
            rdma = pltpu.make_async_remote_copy(
                src_ref=xs.at[d - 1],
                dst_ref=xg.at[3 - d],
                send_sem=send_sems.at[d - 1],
                recv_sem=recv_sems.at[3 - d],
                device_id=((my + d) % N_DEV,),
                device_id_type=pl.DeviceIdType.MESH,
            )
            rdma.start()
            sends.append(rdma)

        stage_steps = {
            0: lambda: (stage_finish(1), stage_start(3)),
            1: lambda: (stage_finish(3), stage_start(2)),
            2: lambda: stage_finish(2),
        }

        loc.wait()
        stage_start(1)

        local_amax = jnp.float32(0.0)
        for f, (o, t) in enumerate(flat):
            if f + 2 < len(flat):
                w_dma(f + 2).start()
            if t == 0 and o != 0:
                with _scope(f"wait_recv#o={o}"):
                    recv = pltpu.make_async_remote_copy(
                        src_ref=xs.at[0],
                        dst_ref=xg.at[o - 1],
                        send_sem=send_sems.at[o - 1],
                        recv_sem=recv_sems.at[o - 1],
                        device_id=(my,),
                        device_id_type=pl.DeviceIdType.MESH,
                    )
                    recv.wait_recv()
                xb[:, :] = xg[o - 1].astype(jnp.float32)
            with _scope(f"mm#o={o}_t={t}"):
                w_dma(f).wait()
                lhs = xf[:, :] if o == 0 else xb[:, :]
                val = jnp.dot(lhs, wbuf[f % WDEPTH],
                              preferred_element_type=jnp.float32)
                if o == 0:
                    acc[:, pl.ds(t * NT, NT)] = val
                else:
                    val = acc[:, pl.ds(t * NT, NT)] + val
                    acc[:, pl.ds(t * NT, NT)] = val
                if o == os_order[-1]:
                    local_amax = jnp.maximum(
                        local_amax, jnp.max(jnp.abs(val)))
            if o == 0 and t in stage_steps:
                stage_steps[t]()

        with _scope("amax_exchange"):
            amax_my[:, :] = jnp.full((8, 128), local_amax, jnp.float32)
            amax_all[0] = amax_my[:, :]
            a_sends = []
            for d in (1, 3, 2):
                rdma = pltpu.make_async_remote_copy(
                    src_ref=amax_my,
                    dst_ref=amax_all.at[N_DEV - d],
                    send_sem=a_send_sems.at[d],
                    recv_sem=a_recv_sems.at[N_DEV - d],
                    device_id=((my + d) % N_DEV,),
                    device_id_type=pl.DeviceIdType.MESH,
                )
                rdma.start()
                a_sends.append(rdma)
            for o in (1, 2, 3):
                recv = pltpu.make_async_remote_copy(
                    src_ref=amax_my,
                    dst_ref=amax_all.at[o],
                    send_sem=a_send_sems.at[o],
                    recv_sem=a_recv_sems.at[o],
                    device_id=(my,),
                    device_id_type=pl.DeviceIdType.MESH,
                )
                recv.wait_recv()
            g_amax = jnp.max(amax_all[:, :, :])

        with _scope("qdq"):
            scale = g_amax / 127.0
            inv = 1.0 / scale
            out_dmas = []
            for t in range(QTN):
                y = acc[:, pl.ds(t * QT, QT)]
                q = jnp.clip(jnp.round(y * inv), -127.0, 127.0)
                acc[:, pl.ds(t * QT, QT)] = q * scale
                if t >= 2:
                    out_dmas[t - 2].wait()
                dma = pltpu.make_async_copy(
                    acc.at[:, pl.ds(t * QT, QT)],
                    out_hbm.at[:, pl.ds(t * QT, QT)],
                    out_sems.at[t % 2])
                dma.start()
                out_dmas.append(dma)
            for dma in out_dmas[-2:]:
                dma.wait()

        for rdma in sends:
            rdma.wait_send()
        for rdma in a_sends:
            rdma.wait_send()

    return pl.pallas_call(
        body,
        out_shape=jax.ShapeDtypeStruct((M_BLK, N_OUT), jnp.float32),
        in_specs=[
            pl.BlockSpec(memory_space=pl.ANY),
            pl.BlockSpec(memory_space=pl.ANY),
        ],
        out_specs=pl.BlockSpec(memory_space=pl.ANY),
        scratch_shapes=[
            pltpu.VMEM((M_BLK, N_OUT), jnp.float32),
            pltpu.VMEM((M_BLK, K_BLK), jnp.float32),
            pltpu.VMEM((N_DEV - 1, M_BLK, K_BLK), jnp.bfloat16),
            pltpu.VMEM((N_DEV - 1, M_BLK, K_BLK), jnp.bfloat16),
            pltpu.VMEM((M_BLK, K_BLK), jnp.float32),
            pltpu.VMEM((WDEPTH, K_BLK, NT), jnp.float32),
            pltpu.VMEM((8, 128), jnp.float32),
            pltpu.VMEM((N_DEV, 8, 128), jnp.float32),
            pltpu.SemaphoreType.DMA((N_DEV,)),
            pltpu.SemaphoreType.DMA((N_DEV,)),
            pltpu.SemaphoreType.DMA((N_DEV,)),
            pltpu.SemaphoreType.DMA((N_DEV,)),
            pltpu.SemaphoreType.DMA((WDEPTH,)),
            pltpu.SemaphoreType.DMA((2,)),
            pltpu.SemaphoreType.DMA,
            pltpu.SemaphoreType.DMA,
        ],
        compiler_params=pltpu.CompilerParams(
            collective_id=0,
            vmem_limit_bytes=128 * 1024 * 1024,
        ),
    )(x, w_mat)


# device time: 141733 ns/iter; 1.0168x vs baseline; 1.0168x over previous
import contextlib
import os

import jax
import jax.numpy as jnp
from jax import lax
from jax.experimental import pallas as pl
from jax.experimental.pallas import tpu as pltpu

PROFILE_SCOPES = False
ABLATE = os.environ.get("SCBAND_ABLATE", "")


def _scope(name):
    return jax.named_scope(name) if PROFILE_SCOPES else contextlib.nullcontext()


N_DEV = 4
M_BLK = 1024
K_BLK = 1024
N_OUT = 8192
NC = 2
KC = K_BLK // NC
NT = 1024
T = N_OUT // NT
WDEPTH = 3
QT = 1024
QTN = N_OUT // QT


def kernel(x, w_mat):

    def body(x_hbm, w_hbm, out_hbm, acc, xs, xg, xb, wbuf, amax_my,
             amax_all, send_sems, recv_sems, a_send_sems, a_recv_sems,
             w_sems, out_sems, loc_sem, stage_sem):
        my = lax.axis_index("i")

        if ABLATE != "gemm":
            barrier_sem = pltpu.get_barrier_semaphore()
            for d in range(1, N_DEV):
                pl.semaphore_signal(
                    barrier_sem, inc=1,
                    device_id=((my + d) % N_DEV,),
                    device_id_type=pl.DeviceIdType.MESH,
                )
            pl.semaphore_wait(barrier_sem, N_DEV - 1)

        os_order = (0, 3, 1, 2)
        flat = [(o, c, t) for o in os_order for c in range(NC)
                for t in range(T)]

        def w_dma(f):
            o, c, t = flat[f]
            j = (my + o) % N_DEV
            return pltpu.make_async_copy(
                w_hbm.at[pl.ds(j * K_BLK + c * KC, KC), pl.ds(t * NT, NT)],
                wbuf.at[f % WDEPTH], w_sems.at[f % WDEPTH])

        if ABLATE != "comm":
            w_dma(0).start()
            w_dma(1).start()
        loc = pltpu.make_async_copy(
            x_hbm.at[pl.ds(my * M_BLK, M_BLK), :], xb, loc_sem)
        loc.start()

        bounce = acc.at[:, pl.ds((T - 1) * NT, K_BLK)]
        sends = []
        if ABLATE != "gemm":
            for d in (1, 3, 2):
                r = (my + d) % N_DEV
                cp = pltpu.make_async_copy(
                    x_hbm.at[pl.ds(r * M_BLK, M_BLK), :], bounce, stage_sem)
                cp.start()
                cp.wait()
                xs[d - 1] = acc[:, pl.ds((T - 1) * NT, K_BLK)].astype(
                    jnp.bfloat16)
                for c in range(NC):
                    rdma = pltpu.make_async_remote_copy(
                        src_ref=xs.at[d - 1, :, pl.ds(c * KC, KC)],
                        dst_ref=xg.at[3 - d, :, pl.ds(c * KC, KC)],
                        send_sem=send_sems.at[d - 1, c],
                        recv_sem=recv_sems.at[3 - d, c],
                        device_id=(r,),
                        device_id_type=pl.DeviceIdType.MESH,
                    )
                    rdma.start()
                    sends.append(rdma)

        loc.wait()

        local_amax = jnp.float32(0.0)
        for f, (o, c, t) in enumerate(flat):
            if t == 0 and o != 0 and ABLATE != "gemm":
                with _scope(f"wait_recv#o={o}_c={c}"):
                    recv = pltpu.make_async_remote_copy(
                        src_ref=xs.at[0, :, pl.ds(c * KC, KC)],
                        dst_ref=xg.at[o - 1, :, pl.ds(c * KC, KC)],
                        send_sem=send_sems.at[o - 1, c],
                        recv_sem=recv_sems.at[o - 1, c],
                        device_id=(my,),
                        device_id_type=pl.DeviceIdType.MESH,
                    )
                    recv.wait_recv()
                xb[:, pl.ds(c * KC, KC)] = (
                    xg[o - 1, :, pl.ds(c * KC, KC)].astype(jnp.float32))
            with _scope(f"mm#o={o}_c={c}_t={t}"):
                if ABLATE != "comm":
                    if f + 2 < len(flat):
                        w_dma(f + 2).start()
                    w_dma(f).wait()
                    lhs = xb[:, pl.ds(c * KC, KC)]
                    val = jnp.dot(lhs, wbuf[f % WDEPTH],
                                  preferred_element_type=jnp.float32)
                    if o == 0 and c == 0:
                        acc[:, pl.ds(t * NT, NT)] = val
                    else:
                        val = acc[:, pl.ds(t * NT, NT)] + val
                        acc[:, pl.ds(t * NT, NT)] = val
                    if o == os_order[-1] and c == NC - 1:
                        local_amax = jnp.maximum(
                            local_amax, jnp.max(jnp.abs(val)))

        if ABLATE == "gemm":
            g_amax = local_amax
            a_sends = []
        else:
          with _scope("amax_exchange"):
            amax_my[:, :] = jnp.full((8, 128), local_amax, jnp.float32)
            amax_all[0] = amax_my[:, :]
            a_sends = []
            for d in (1, 3, 2):
                rdma = pltpu.make_async_remote_copy(
                    src_ref=amax_my,
                    dst_ref=amax_all.at[N_DEV - d],
                    send_sem=a_send_sems.at[d],
                    recv_sem=a_recv_sems.at[N_DEV - d],
                    device_id=((my + d) % N_DEV,),
                    device_id_type=pl.DeviceIdType.MESH,
                )
                rdma.start()
                a_sends.append(rdma)
            for o in (1, 2, 3):
                recv = pltpu.make_async_remote_copy(
                    src_ref=amax_my,
                    dst_ref=amax_all.at[o],
                    send_sem=a_send_sems.at[o],
                    recv_sem=a_recv_sems.at[o],
                    device_id=(my,),
                    device_id_type=pl.DeviceIdType.MESH,
                )
                recv.wait_recv()
            g_amax = jnp.max(amax_all[:, :, :])

        with _scope("qdq"):
            scale = g_amax / 127.0
            inv = 1.0 / scale
            out_dmas = []
            for t in range(QTN):
                y = acc[:, pl.ds(t * QT, QT)]
                q = jnp.clip(jnp.round(y * inv), -127.0, 127.0)
                acc[:, pl.ds(t * QT, QT)] = q * scale
                if t >= 2:
                    out_dmas[t - 2].wait()
                dma = pltpu.make_async_copy(
                    acc.at[:, pl.ds(t * QT, QT)],
                    out_hbm.at[:, pl.ds(t * QT, QT)],
                    out_sems.at[t % 2])
                dma.start()
                out_dmas.append(dma)
            for dma in out_dmas[-2:]:
                dma.wait()

        for rdma in sends:
            rdma.wait_send()
        for rdma in a_sends:
            rdma.wait_send()

    return pl.pallas_call(
        body,
        out_shape=jax.ShapeDtypeStruct((M_BLK, N_OUT), jnp.float32),
        in_specs=[
            pl.BlockSpec(memory_space=pl.ANY),
            pl.BlockSpec(memory_space=pl.ANY),
        ],
        out_specs=pl.BlockSpec(memory_space=pl.ANY),
        scratch_shapes=[
            pltpu.VMEM((M_BLK, N_OUT), jnp.float32),
            pltpu.VMEM((N_DEV - 1, M_BLK, K_BLK), jnp.bfloat16),
            pltpu.VMEM((N_DEV - 1, M_BLK, K_BLK), jnp.bfloat16),
            pltpu.VMEM((M_BLK, K_BLK), jnp.float32),
            pltpu.VMEM((WDEPTH, KC, NT), jnp.float32),
            pltpu.VMEM((8, 128), jnp.float32),
            pltpu.VMEM((N_DEV, 8, 128), jnp.float32),
            pltpu.SemaphoreType.DMA((N_DEV - 1, NC)),
            pltpu.SemaphoreType.DMA((N_DEV - 1, NC)),
            pltpu.SemaphoreType.DMA((N_DEV,)),
            pltpu.SemaphoreType.DMA((N_DEV,)),
            pltpu.SemaphoreType.DMA((WDEPTH,)),
            pltpu.SemaphoreType.DMA((2,)),
            pltpu.SemaphoreType.DMA,
            pltpu.SemaphoreType.DMA,
        ],
        compiler_params=pltpu.CompilerParams(
            collective_id=None if ABLATE == "gemm" else 0,
            vmem_limit_bytes=128 * 1024 * 1024,
        ),
    )(x, w_mat)
